# baseline (speedup 1.0000x reference)
import jax
import jax.numpy as jnp
from jax import lax
from jax.experimental import pallas as pl
from jax.experimental.pallas import tpu as pltpu
from jax.experimental.pallas import tpu_sc as plsc

N = 10000
E = 160000
D_IN = 512
D = 256
DH = 128
NG = 32
NA = 2000
NH = 4000
NUM_ACT = 20
NUM_SACT = 91
NUM_AACT = 17
NUM_ACTOR = 26
NUM_RELAT = 19

NTEC = 16
NCORE = 2
CE_ = 40
NCH_MSG = 250
BCH_MSG = 50
CN = 80
NCH_POOL = 125
CR = 100
NCH_REL = 50
PJ = 32

_mesh = plsc.VectorSubcoreMesh(core_axis_name="c", subcore_axis_name="s",
                               num_cores=NCORE, num_subcores=NTEC)


def _node_mlp_body(x_ref, w1_ref, b1_ref, w2_ref, b2_ref, out_ref):
    h = jnp.maximum(jnp.dot(x_ref[...], w1_ref[...],
                            preferred_element_type=jnp.float32) + b1_ref[...], 0.0)
    res = jnp.dot(h, w2_ref[...], preferred_element_type=jnp.float32) + b2_ref[...]
    out_ref[0] = res[:, :DH]
    out_ref[1] = res[:, DH:]


def _node_mlp(x, w1, b1, w2, b2):
    bn = 400
    grid = N // bn
    return pl.pallas_call(
        _node_mlp_body,
        grid=(grid,),
        in_specs=[
            pl.BlockSpec((bn, D_IN), lambda i: (i, 0)),
            pl.BlockSpec((D_IN, D), lambda i: (0, 0)),
            pl.BlockSpec((1, D), lambda i: (0, 0)),
            pl.BlockSpec((D, D), lambda i: (0, 0)),
            pl.BlockSpec((1, D), lambda i: (0, 0)),
        ],
        out_specs=pl.BlockSpec((NCORE, bn, DH), lambda i: (0, i, 0)),
        out_shape=jax.ShapeDtypeStruct((NCORE, N, DH), jnp.float32),
    )(x, w1, b1.reshape(1, D), w2, b2.reshape(1, D))


def _edge_mlp_body(e_ref, w1_ref, b1_ref, w2_ref, b2_ref, out_ref):
    h = jnp.maximum(jnp.dot(e_ref[...], w1_ref[...],
                            preferred_element_type=jnp.float32) + b1_ref[...], 0.0)
    res = jnp.dot(h, w2_ref[...], preferred_element_type=jnp.float32) + b2_ref[...]
    out_ref[0] = res[:, :DH]
    out_ref[1] = res[:, DH:]


def _edge_mlp(ea, w1, b1, w2, b2):
    be = 2000
    grid = E // be
    return pl.pallas_call(
        _edge_mlp_body,
        grid=(grid,),
        in_specs=[
            pl.BlockSpec((be, 9), lambda i: (i, 0)),
            pl.BlockSpec((9, D), lambda i: (0, 0)),
            pl.BlockSpec((1, D), lambda i: (0, 0)),
            pl.BlockSpec((D, D), lambda i: (0, 0)),
            pl.BlockSpec((1, D), lambda i: (0, 0)),
        ],
        out_specs=pl.BlockSpec((NCORE, be, DH), lambda i: (0, i, 0)),
        out_shape=jax.ShapeDtypeStruct((NCORE, E, DH), jnp.float32),
    )(ea, w1, b1.reshape(1, D), w2, b2.reshape(1, D))


def _layer_body(n_ref, a_ref, w_ref, b_ref, out_ref):
    a0 = n_ref[0] + a_ref[0]
    a1 = n_ref[1] + a_ref[1]
    res = (jnp.dot(a0, w_ref[0:DH, :], preferred_element_type=jnp.float32)
           + jnp.dot(a1, w_ref[DH:, :], preferred_element_type=jnp.float32)
           + b_ref[...])
    res = jnp.maximum(res, 0.0)
    out_ref[0] = res[:, :DH]
    out_ref[1] = res[:, DH:]


def _layer(node_st, agg_st, w, b):
    bn = 400
    grid = N // bn
    return pl.pallas_call(
        _layer_body,
        grid=(grid,),
        in_specs=[
            pl.BlockSpec((NCORE, bn, DH), lambda i: (0, i, 0)),
            pl.BlockSpec((NCORE, bn, DH), lambda i: (0, i, 0)),
            pl.BlockSpec((D, D), lambda i: (0, 0)),
            pl.BlockSpec((1, D), lambda i: (0, 0)),
        ],
        out_specs=pl.BlockSpec((NCORE, bn, DH), lambda i: (0, i, 0)),
        out_shape=jax.ShapeDtypeStruct((NCORE, N, DH), jnp.float32),
    )(node_st, agg_st, w, b.reshape(1, D))


def _layer2_body(n_ref, a_ref, w_ref, b_ref, wtb_ref, c_ref,
                 out_ref, pc_ref):
    a0 = n_ref[0] + a_ref[0]
    a1 = n_ref[1] + a_ref[1]
    res = (jnp.dot(a0, w_ref[0:DH, :], preferred_element_type=jnp.float32)
           + jnp.dot(a1, w_ref[DH:, :], preferred_element_type=jnp.float32)
           + b_ref[...])
    res = jnp.maximum(res, 0.0)
    out_ref[0] = res[:, :DH]
    out_ref[1] = res[:, DH:]
    pc_ref[...] = jnp.dot(res, wtb_ref[...], preferred_element_type=jnp.float32) + c_ref[...]


def _layer2(node_st, agg_st, w, b, wtb, crow):
    bn = 400
    grid = N // bn
    return pl.pallas_call(
        _layer2_body,
        grid=(grid,),
        in_specs=[
            pl.BlockSpec((NCORE, bn, DH), lambda i: (0, i, 0)),
            pl.BlockSpec((NCORE, bn, DH), lambda i: (0, i, 0)),
            pl.BlockSpec((D, D), lambda i: (0, 0)),
            pl.BlockSpec((1, D), lambda i: (0, 0)),
            pl.BlockSpec((D, DH), lambda i: (0, 0)),
            pl.BlockSpec((1, DH), lambda i: (0, 0)),
        ],
        out_specs=[
            pl.BlockSpec((NCORE, bn, DH), lambda i: (0, i, 0)),
            pl.BlockSpec((bn, DH), lambda i: (i, 0)),
        ],
        out_shape=[
            jax.ShapeDtypeStruct((NCORE, N, DH), jnp.float32),
            jax.ShapeDtypeStruct((N, DH), jnp.float32),
        ],
    )(node_st, agg_st, w, b.reshape(1, D), wtb, crow)


def _counts_body(ba_ref, b_ref, ca_ref, cg_ref):
    i = pl.program_id(0)
    oh_a = (ba_ref[...] == lax.broadcasted_iota(jnp.int32, (200, NA), 1)
            ).astype(jnp.float32)
    oh_g = (b_ref[...] == lax.broadcasted_iota(jnp.int32, (200, NG), 1)
            ).astype(jnp.float32)
    pa = jnp.sum(oh_a, axis=0, keepdims=True)
    pg = jnp.sum(oh_g, axis=0, keepdims=True)

    @pl.when(i == 0)
    def _():
        ca_ref[...] = pa
        cg_ref[...] = pg

    @pl.when(i > 0)
    def _():
        ca_ref[...] = ca_ref[...] + pa
        cg_ref[...] = cg_ref[...] + pg


def _counts(batch_actor, batch):
    grid = N // 200
    return pl.pallas_call(
        _counts_body,
        grid=(grid,),
        in_specs=[
            pl.BlockSpec((200, 1), lambda i: (i, 0)),
            pl.BlockSpec((200, 1), lambda i: (i, 0)),
        ],
        out_specs=[
            pl.BlockSpec((1, NA), lambda i: (0, 0)),
            pl.BlockSpec((1, NG), lambda i: (0, 0)),
        ],
        out_shape=[
            jax.ShapeDtypeStruct((1, NA), jnp.float32),
            jax.ShapeDtypeStruct((1, NG), jnp.float32),
        ],
    )(batch_actor.reshape(N, 1), batch.reshape(N, 1))


def _striped(total):
    per = -(-total // NTEC)
    per = -(-per // 8) * 8
    last = total - 15 * per
    assert last > 0 and last % 8 == 0
    return per, last


SZ_N, SZ_N_LAST = _striped(N)
SZ_A, SZ_A_LAST = _striped(NA)
SZ_R, SZ_R_LAST = _striped(NH)


def _msg_body(node_flat, edge_flat, src_idx, dst_idx, z128, agg_out,
              src_v, dst_v, ebuf, gbuf, ebuf1, gbuf1, agg_s,
              se0, sg0, se1, sg1, ss0, ss1):
    c = lax.axis_index("c")
    s = lax.axis_index("s")

    @pl.when(s < 15)
    def _():
        pltpu.sync_copy(z128.at[pl.ds(0, SZ_N), :],
                        agg_s.at[pl.ds(s * SZ_N, SZ_N), :])

    @pl.when(s == 15)
    def _():
        pltpu.sync_copy(z128.at[pl.ds(0, SZ_N_LAST), :],
                        agg_s.at[pl.ds(15 * SZ_N, SZ_N_LAST), :])

    plsc.subcore_barrier()

    def estart(o, j, eb, sem):
        e0 = s * (NCH_MSG * CE_) + (o * BCH_MSG + j) * CE_
        pltpu.async_copy(edge_flat.at[pl.ds(c * E + e0, CE_), :], eb, sem)

    def gstart(j, gb, sem):
        pltpu.async_copy(node_flat.at[src_v.at[j]], gb, sem)

    def ewait(eb, sem):
        pltpu.make_async_copy(edge_flat.at[pl.ds(0, CE_), :], eb, sem).wait()

    def gwait(j, gb, sem):
        pltpu.make_async_copy(node_flat.at[src_v.at[j]], gb, sem).wait()

    def compute(eb, gb):
        @plsc.parallel_loop(0, CE_, step=1, unroll=2)
        def _row(r):
            for k in range(DH // 16):
                sl = pl.ds(k * 16, 16)
                gb[r, sl] = jnp.maximum(gb[r, sl] + eb[r, sl], 0.0)

    def sstart(j, gb, sem):
        pltpu.async_copy(gb, agg_s.at[dst_v.at[j]], sem, add=True)

    def swait(j, gb, sem):
        pltpu.make_async_copy(gb, agg_s.at[dst_v.at[j]], sem).wait()

    def block(o, carry0):
        pltpu.sync_copy(src_idx.at[c, s, o], src_v)
        pltpu.sync_copy(dst_idx.at[s, o], dst_v)

        estart(o, 0, ebuf, se0)
        gstart(0, gbuf, sg0)

        def pair(j2, carry):
            j0 = 2 * j2
            j1 = j0 + 1

            @pl.when(j2 > 0)
            def _():
                swait(j1 - 2, gbuf1, ss1)

            estart(o, j1, ebuf1, se1)
            gstart(j1, gbuf1, sg1)

            ewait(ebuf, se0)
            gwait(j0, gbuf, sg0)
            compute(ebuf, gbuf)
            sstart(j0, gbuf, ss0)

            ewait(ebuf1, se1)
            gwait(j1, gbuf1, sg1)
            compute(ebuf1, gbuf1)
            sstart(j1, gbuf1, ss1)

            @pl.when(j2 + 1 < BCH_MSG // 2)
            def _():
                swait(j0, gbuf, ss0)
                estart(o, j0 + 2, ebuf, se0)
                gstart(j0 + 2, gbuf, sg0)

            return carry

        lax.fori_loop(0, BCH_MSG // 2, pair, 0, unroll=False)
        swait(BCH_MSG - 2, gbuf, ss0)
        swait(BCH_MSG - 1, gbuf1, ss1)
        return carry0

    lax.fori_loop(0, NCH_MSG // BCH_MSG, block, 0, unroll=False)

    plsc.subcore_barrier()

    @pl.when(s < 15)
    def _():
        pltpu.sync_copy(agg_s.at[pl.ds(s * SZ_N, SZ_N), :],
                        agg_out.at[c, pl.ds(s * SZ_N, SZ_N), :])

    @pl.when(s == 15)
    def _():
        pltpu.sync_copy(agg_s.at[pl.ds(15 * SZ_N, SZ_N_LAST), :],
                        agg_out.at[c, pl.ds(15 * SZ_N, SZ_N_LAST), :])


def _sc_msg(node_flat, edge_flat, src_idx2, dst_idx, z128):
    f = pl.kernel(
        _msg_body,
        out_type=jax.ShapeDtypeStruct((NCORE, N, DH), jnp.float32),
        mesh=_mesh,
        scratch_types=[
            pltpu.VMEM((BCH_MSG, CE_), jnp.int32),
            pltpu.VMEM((BCH_MSG, CE_), jnp.int32),
            pltpu.VMEM((CE_, DH), jnp.float32),
            pltpu.VMEM((CE_, DH), jnp.float32),
            pltpu.VMEM((CE_, DH), jnp.float32),
            pltpu.VMEM((CE_, DH), jnp.float32),
            pltpu.VMEM_SHARED((N, DH), jnp.float32),
            pltpu.SemaphoreType.DMA,
            pltpu.SemaphoreType.DMA,
            pltpu.SemaphoreType.DMA,
            pltpu.SemaphoreType.DMA,
            pltpu.SemaphoreType.DMA,
            pltpu.SemaphoreType.DMA,
        ],
    )
    return f(node_flat, edge_flat, src_idx2, dst_idx, z128)


def _pool_body(node_flat, pc, ba_idx, b_idx, src_idx, dst_idx, hyp_idx,
               z128,
               a_sum, g_sum, r_part,
               ba_v, b_v, nbuf, srcv, dstv, hypv, pbuf, qbuf,
               actor_s, g_s, rel_s):
    c = lax.axis_index("c")
    s = lax.axis_index("s")
    w = c * NTEC + s

    @pl.when(s < 15)
    def _():
        pltpu.sync_copy(z128.at[pl.ds(0, SZ_A), :],
                        actor_s.at[pl.ds(s * SZ_A, SZ_A), :])
        pltpu.sync_copy(z128.at[pl.ds(0, SZ_R), :],
                        rel_s.at[pl.ds(s * SZ_R, SZ_R), :])

    @pl.when(s == 15)
    def _():
        pltpu.sync_copy(z128.at[pl.ds(0, SZ_A_LAST), :],
                        actor_s.at[pl.ds(15 * SZ_A, SZ_A_LAST), :])
        pltpu.sync_copy(z128.at[pl.ds(0, SZ_R_LAST), :],
                        rel_s.at[pl.ds(15 * SZ_R, SZ_R_LAST), :])

    @pl.when(s == 0)
    def _():
        pltpu.sync_copy(z128.at[pl.ds(0, NG), :], g_s)

    pltpu.sync_copy(ba_idx.at[pl.ds(s * 8, 8), :], ba_v)
    pltpu.sync_copy(b_idx.at[pl.ds(s * 8, 8), :], b_v)
    pltpu.sync_copy(src_idx.at[w], srcv)
    pltpu.sync_copy(dst_idx.at[w], dstv)
    pltpu.sync_copy(hyp_idx.at[w], hypv)

    plsc.subcore_barrier()

    def nchunk(j, carry):
        q = s * 8 + j

        @pl.when(q < NCH_POOL)
        def _():
            n0 = c * N + q * CN
            pltpu.sync_copy(node_flat.at[pl.ds(n0, CN), :], nbuf)
            pltpu.sync_copy(nbuf, actor_s.at[ba_v.at[j]], add=True)
            pltpu.sync_copy(nbuf, g_s.at[b_v.at[j]], add=True)

        return carry

    lax.fori_loop(0, 8, nchunk, 0, unroll=False)

    def rchunk(j, carry):
        pltpu.sync_copy(pc.at[srcv.at[j]], pbuf)
        pltpu.sync_copy(pc.at[dstv.at[j]], qbuf)

        @plsc.parallel_loop(0, CR, step=1, unroll=2)
        def _row(r):
            for k in range(PJ // 16):
                pbuf[r, pl.ds(k * 16, 16)] = (pbuf[r, pl.ds(k * 16, 16)]
                                              + qbuf[r, pl.ds(PJ + k * 16, 16)])
        pltpu.sync_copy(pbuf, rel_s.at[hypv.at[j]], add=True)
        return carry

    lax.fori_loop(0, NCH_REL, rchunk, 0, unroll=False)

    plsc.subcore_barrier()

    @pl.when(s < 15)
    def _():
        pltpu.sync_copy(actor_s.at[pl.ds(s * SZ_A, SZ_A), :],
                        a_sum.at[c, pl.ds(s * SZ_A, SZ_A), :])
        pltpu.sync_copy(rel_s.at[pl.ds(s * SZ_R, SZ_R), :],
                        r_part.at[c, pl.ds(s * SZ_R, SZ_R), :])

    @pl.when(s == 15)
    def _():
        pltpu.sync_copy(actor_s.at[pl.ds(15 * SZ_A, SZ_A_LAST), :],
                        a_sum.at[c, pl.ds(15 * SZ_A, SZ_A_LAST), :])
        pltpu.sync_copy(rel_s.at[pl.ds(15 * SZ_R, SZ_R_LAST), :],
                        r_part.at[c, pl.ds(15 * SZ_R, SZ_R_LAST), :])

    @pl.when(s == 0)
    def _():
        pltpu.sync_copy(g_s, g_sum.at[c])


def _sc_pool(node_flat, pc, ba_idx, b_idx, src_idx, dst_idx, hyp_idx, z128):
    f = pl.kernel(
        _pool_body,
        out_type=[
            jax.ShapeDtypeStruct((NCORE, NA, DH), jnp.float32),
            jax.ShapeDtypeStruct((NCORE, NG, DH), jnp.float32),
            jax.ShapeDtypeStruct((NCORE, NH, DH), jnp.float32),
        ],
        mesh=_mesh,
        scratch_types=[
            pltpu.VMEM((8, CN), jnp.int32),
            pltpu.VMEM((8, CN), jnp.int32),
            pltpu.VMEM((CN, DH), jnp.float32),
            pltpu.VMEM((NCH_REL, CR), jnp.int32),
            pltpu.VMEM((NCH_REL, CR), jnp.int32),
            pltpu.VMEM((NCH_REL, CR), jnp.int32),
            pltpu.VMEM((CR, DH), jnp.float32),
            pltpu.VMEM((CR, DH), jnp.float32),
            pltpu.VMEM_SHARED((NA, DH), jnp.float32),
            pltpu.VMEM_SHARED((NG, DH), jnp.float32),
            pltpu.VMEM_SHARED((NH, DH), jnp.float32),
        ],
    )
    return f(node_flat, pc, ba_idx, b_idx, src_idx, dst_idx, hyp_idx, z128)


def _heads_kernel(a_sum, a_cnt, g_sum, g_cnt, r_part,
                  act_cids, sact_cids, ps_t, pa_t, actor_cids, h_label,
                  W_act, b_act, W_sact, b_sact, W_ps, b_ps, W_pa, b_pa,
                  W_actor, b_actor, b_relat, out):
    embed = a_sum[...] / jnp.clip(a_cnt[...], 1.0)
    g = g_sum[...] / jnp.clip(g_cnt[...], 1.0)

    def ce(logits, labels_col):
        m = jnp.max(logits, axis=-1, keepdims=True)
        z = logits - m
        logp = z - jnp.log(jnp.sum(jnp.exp(z), axis=-1, keepdims=True))
        iot = lax.broadcasted_iota(jnp.int32, logits.shape, 1)
        onehot = (iot == labels_col).astype(jnp.float32)
        return -jnp.sum(logp * onehot) / logits.shape[0]

    def bce(logits, t):
        v = jnp.clip(logits, 0.0) - logits * t + jnp.log(1.0 + jnp.exp(-jnp.abs(logits)))
        return jnp.sum(v) / (v.shape[0] * v.shape[1])

    logits_act = g @ W_act[...] + b_act[...]
    logits_sact = g @ W_sact[...] + b_sact[...]
    logits_ps = g @ W_ps[...] + b_ps[...]
    logits_pa = embed @ W_pa[...] + b_pa[...]
    logits_actor = embed @ W_actor[...] + b_actor[...]

    rp = r_part[...]
    rs = rp[0] + rp[1]
    rc = rs[:, NUM_RELAT:NUM_RELAT + 1]
    logits_relat = jnp.where(rc > 0.0,
                             rs[:, 0:NUM_RELAT] / jnp.clip(rc, 1.0) + b_relat[...],
                             0.0)

    loss = (ce(logits_act, act_cids[...])
            + ce(logits_sact, sact_cids[...])
            + bce(logits_ps, ps_t[...])
            + bce(logits_pa, pa_t[...])
            + ce(logits_actor, actor_cids[...])
            + ce(logits_relat, h_label[...]))
    out[...] = jnp.reshape(loss, (1, 1))


def _heads_loss(a_sum, a_cnt, g_sum, g_cnt, r_part,
                act_cids, sact_cids, ps_t, pa_t, actor_cids, h_label, p):
    args = (a_sum, a_cnt, g_sum, g_cnt, r_part,
            act_cids.reshape(NG, 1), sact_cids.reshape(NG, 1), ps_t, pa_t,
            actor_cids.reshape(NA, 1), h_label.reshape(NH, 1),
            p['W_act'], p['b_act'].reshape(1, -1),
            p['W_sact'], p['b_sact'].reshape(1, -1),
            p['W_ps'], p['b_ps'].reshape(1, -1),
            p['W_pa'], p['b_pa'].reshape(1, -1),
            p['W_actor'], p['b_actor'].reshape(1, -1),
            p['b_relat'].reshape(1, -1))
    out = pl.pallas_call(
        _heads_kernel,
        out_shape=jax.ShapeDtypeStruct((1, 1), jnp.float32),
    )(*args)
    return out.reshape(())


def kernel(x, edge_attr, edge_index, batch, batch_actor, hyperedge_ids,
           act_cids, sact_cids, ps_aact_cids, pa_aact_cids, actor_cids,
           hyperedge_label, params):
    p = params
    src = edge_index[0]
    dst = edge_index[1]

    src16 = src.reshape(NTEC, NCH_MSG // BCH_MSG, BCH_MSG, CE_)
    src16_2 = jnp.stack([src16, src16 + N])
    dst16 = dst.reshape(NTEC, NCH_MSG // BCH_MSG, BCH_MSG, CE_)
    src32 = src.reshape(NCORE * NTEC, NCH_REL, CR)
    dst32 = dst.reshape(NCORE * NTEC, NCH_REL, CR)
    hyp32 = hyperedge_ids.reshape(NCORE * NTEC, NCH_REL, CR)
    ba_idx = jnp.pad(batch_actor.reshape(NCH_POOL, CN), ((0, 3), (0, 0)))
    b_idx = jnp.pad(batch.reshape(NCH_POOL, CN), ((0, 3), (0, 0)))
    z128 = jnp.zeros((SZ_N, DH), jnp.float32)

    wtb = (jnp.zeros((D, DH), jnp.float32)
           .at[:, :NUM_RELAT].set(p['W_relat'][:D])
           .at[:, PJ:PJ + NUM_RELAT].set(p['W_relat'][D:]))
    crow = (jnp.zeros((1, DH), jnp.float32)
            .at[0, NUM_RELAT].set(0.5)
            .at[0, PJ + NUM_RELAT].set(0.5))

    node_st = _node_mlp(x, p['Wn1'], p['bn1'], p['Wn2'], p['bn2'])
    edge_st = _edge_mlp(edge_attr, p['We1'], p['be1'], p['We2'], p['be2'])
    node_flat = node_st.reshape(NCORE * N, DH)
    edge_flat = edge_st.reshape(NCORE * E, DH)

    agg1 = _sc_msg(node_flat, edge_flat, src16_2, dst16, z128)
    node_st = _layer(node_st, agg1, p['Wl1'], p['bl1'])
    node_flat = node_st.reshape(NCORE * N, DH)

    agg2 = _sc_msg(node_flat, edge_flat, src16_2, dst16, z128)
    node_st, pc = _layer2(node_st, agg2, p['Wl2'], p['bl2'], wtb, crow)
    node_flat = node_st.reshape(NCORE * N, DH)

    ca, cg = _counts(batch_actor, batch)
    a_sum, g_sum, r_part = _sc_pool(
        node_flat, pc, ba_idx, b_idx, src32, dst32, hyp32, z128)

    a_full = jnp.concatenate([a_sum[0], a_sum[1]], axis=1)
    g_full = jnp.concatenate([g_sum[0], g_sum[1]], axis=1)

    return _heads_loss(a_full, ca.reshape(NA, 1), g_full, cg.reshape(NG, 1),
                       r_part, act_cids, sact_cids, ps_aact_cids,
                       pa_aact_cids, actor_cids, hyperedge_label, p)


# speedup vs baseline: 1.1612x; 1.1612x over previous
import jax
import jax.numpy as jnp
from jax import lax
from jax.experimental import pallas as pl
from jax.experimental.pallas import tpu as pltpu
from jax.experimental.pallas import tpu_sc as plsc

N = 10000
E = 160000
D_IN = 512
D = 256
DH = 128
NG = 32
NA = 2000
NH = 4000
NUM_ACT = 20
NUM_SACT = 91
NUM_AACT = 17
NUM_ACTOR = 26
NUM_RELAT = 19

NTEC = 16
NCORE = 2
CE_ = 40
NCH_MSG = 250
BCH_MSG = 50
CN = 80
NCH_POOL = 125
CR = 50
NCH_REL = 100
PJ = 32

_mesh = plsc.VectorSubcoreMesh(core_axis_name="c", subcore_axis_name="s",
                               num_cores=NCORE, num_subcores=NTEC)


def _node_mlp_body(x_ref, w1_ref, b1_ref, w2_ref, b2_ref, out_ref):
    h = jnp.maximum(jnp.dot(x_ref[...], w1_ref[...],
                            preferred_element_type=jnp.float32) + b1_ref[...], 0.0)
    res = jnp.dot(h, w2_ref[...], preferred_element_type=jnp.float32) + b2_ref[...]
    out_ref[0] = res[:, :DH]
    out_ref[1] = res[:, DH:]


def _node_mlp(x, w1, b1, w2, b2):
    bn = 400
    grid = N // bn
    return pl.pallas_call(
        _node_mlp_body,
        grid=(grid,),
        in_specs=[
            pl.BlockSpec((bn, D_IN), lambda i: (i, 0)),
            pl.BlockSpec((D_IN, D), lambda i: (0, 0)),
            pl.BlockSpec((1, D), lambda i: (0, 0)),
            pl.BlockSpec((D, D), lambda i: (0, 0)),
            pl.BlockSpec((1, D), lambda i: (0, 0)),
        ],
        out_specs=pl.BlockSpec((NCORE, bn, DH), lambda i: (0, i, 0)),
        out_shape=jax.ShapeDtypeStruct((NCORE, N, DH), jnp.float32),
    )(x, w1, b1.reshape(1, D), w2, b2.reshape(1, D))


def _edge_mlp_body(e_ref, w1_ref, b1_ref, w2_ref, b2_ref, out_ref):
    h = jnp.maximum(jnp.dot(e_ref[...], w1_ref[...],
                            preferred_element_type=jnp.float32) + b1_ref[...], 0.0)
    res = jnp.dot(h, w2_ref[...], preferred_element_type=jnp.float32) + b2_ref[...]
    out_ref[0] = res[:, :DH]
    out_ref[1] = res[:, DH:]


def _edge_mlp(ea, w1, b1, w2, b2):
    be = 2000
    grid = E // be
    return pl.pallas_call(
        _edge_mlp_body,
        grid=(grid,),
        in_specs=[
            pl.BlockSpec((be, 9), lambda i: (i, 0)),
            pl.BlockSpec((9, D), lambda i: (0, 0)),
            pl.BlockSpec((1, D), lambda i: (0, 0)),
            pl.BlockSpec((D, D), lambda i: (0, 0)),
            pl.BlockSpec((1, D), lambda i: (0, 0)),
        ],
        out_specs=pl.BlockSpec((NCORE, be, DH), lambda i: (0, i, 0)),
        out_shape=jax.ShapeDtypeStruct((NCORE, E, DH), jnp.float32),
    )(ea, w1, b1.reshape(1, D), w2, b2.reshape(1, D))


def _layer_body(n_ref, a_ref, w_ref, b_ref, out_ref):
    a0 = n_ref[0] + a_ref[0]
    a1 = n_ref[1] + a_ref[1]
    res = (jnp.dot(a0, w_ref[0:DH, :], preferred_element_type=jnp.float32)
           + jnp.dot(a1, w_ref[DH:, :], preferred_element_type=jnp.float32)
           + b_ref[...])
    res = jnp.maximum(res, 0.0)
    out_ref[0] = res[:, :DH]
    out_ref[1] = res[:, DH:]


def _layer(node_st, agg_st, w, b):
    bn = 400
    grid = N // bn
    return pl.pallas_call(
        _layer_body,
        grid=(grid,),
        in_specs=[
            pl.BlockSpec((NCORE, bn, DH), lambda i: (0, i, 0)),
            pl.BlockSpec((NCORE, bn, DH), lambda i: (0, i, 0)),
            pl.BlockSpec((D, D), lambda i: (0, 0)),
            pl.BlockSpec((1, D), lambda i: (0, 0)),
        ],
        out_specs=pl.BlockSpec((NCORE, bn, DH), lambda i: (0, i, 0)),
        out_shape=jax.ShapeDtypeStruct((NCORE, N, DH), jnp.float32),
    )(node_st, agg_st, w, b.reshape(1, D))


def _layer2_body(n_ref, a_ref, w_ref, b_ref, wtb_ref, c_ref,
                 out_ref, pc_ref):
    a0 = n_ref[0] + a_ref[0]
    a1 = n_ref[1] + a_ref[1]
    res = (jnp.dot(a0, w_ref[0:DH, :], preferred_element_type=jnp.float32)
           + jnp.dot(a1, w_ref[DH:, :], preferred_element_type=jnp.float32)
           + b_ref[...])
    res = jnp.maximum(res, 0.0)
    out_ref[0] = res[:, :DH]
    out_ref[1] = res[:, DH:]
    pc_ref[...] = jnp.dot(res, wtb_ref[...], preferred_element_type=jnp.float32) + c_ref[...]


def _layer2(node_st, agg_st, w, b, wtb, crow):
    bn = 400
    grid = N // bn
    return pl.pallas_call(
        _layer2_body,
        grid=(grid,),
        in_specs=[
            pl.BlockSpec((NCORE, bn, DH), lambda i: (0, i, 0)),
            pl.BlockSpec((NCORE, bn, DH), lambda i: (0, i, 0)),
            pl.BlockSpec((D, D), lambda i: (0, 0)),
            pl.BlockSpec((1, D), lambda i: (0, 0)),
            pl.BlockSpec((D, DH), lambda i: (0, 0)),
            pl.BlockSpec((1, DH), lambda i: (0, 0)),
        ],
        out_specs=[
            pl.BlockSpec((NCORE, bn, DH), lambda i: (0, i, 0)),
            pl.BlockSpec((bn, DH), lambda i: (i, 0)),
        ],
        out_shape=[
            jax.ShapeDtypeStruct((NCORE, N, DH), jnp.float32),
            jax.ShapeDtypeStruct((N, DH), jnp.float32),
        ],
    )(node_st, agg_st, w, b.reshape(1, D), wtb, crow)


def _counts_body(ba_ref, b_ref, ca_ref, cg_ref):
    i = pl.program_id(0)
    oh_a = (ba_ref[...] == lax.broadcasted_iota(jnp.int32, (200, NA), 1)
            ).astype(jnp.float32)
    oh_g = (b_ref[...] == lax.broadcasted_iota(jnp.int32, (200, NG), 1)
            ).astype(jnp.float32)
    pa = jnp.sum(oh_a, axis=0, keepdims=True)
    pg = jnp.sum(oh_g, axis=0, keepdims=True)

    @pl.when(i == 0)
    def _():
        ca_ref[...] = pa
        cg_ref[...] = pg

    @pl.when(i > 0)
    def _():
        ca_ref[...] = ca_ref[...] + pa
        cg_ref[...] = cg_ref[...] + pg


def _counts(batch_actor, batch):
    grid = N // 200
    return pl.pallas_call(
        _counts_body,
        grid=(grid,),
        in_specs=[
            pl.BlockSpec((200, 1), lambda i: (i, 0)),
            pl.BlockSpec((200, 1), lambda i: (i, 0)),
        ],
        out_specs=[
            pl.BlockSpec((1, NA), lambda i: (0, 0)),
            pl.BlockSpec((1, NG), lambda i: (0, 0)),
        ],
        out_shape=[
            jax.ShapeDtypeStruct((1, NA), jnp.float32),
            jax.ShapeDtypeStruct((1, NG), jnp.float32),
        ],
    )(batch_actor.reshape(N, 1), batch.reshape(N, 1))


def _striped(total):
    per = -(-total // NTEC)
    per = -(-per // 8) * 8
    last = total - 15 * per
    assert last > 0 and last % 8 == 0
    return per, last


SZ_N, SZ_N_LAST = _striped(N)
SZ_A, SZ_A_LAST = _striped(NA)
SZ_R, SZ_R_LAST = _striped(NH)


def _msg_body(node_flat, edge_flat, src_idx, dst_idx, z128, agg_out,
              src_v, dst_v, ebuf, gbuf, ebuf1, gbuf1, agg_s,
              se0, sg0, se1, sg1):
    c = lax.axis_index("c")
    s = lax.axis_index("s")

    @pl.when(s < 15)
    def _():
        pltpu.sync_copy(z128.at[pl.ds(0, SZ_N), :],
                        agg_s.at[pl.ds(s * SZ_N, SZ_N), :])

    @pl.when(s == 15)
    def _():
        pltpu.sync_copy(z128.at[pl.ds(0, SZ_N_LAST), :],
                        agg_s.at[pl.ds(15 * SZ_N, SZ_N_LAST), :])

    plsc.subcore_barrier()

    def estart(o, j, eb, sem):
        e0 = s * (NCH_MSG * CE_) + (o * BCH_MSG + j) * CE_
        pltpu.async_copy(edge_flat.at[pl.ds(c * E + e0, CE_), :], eb, sem)

    def gstart(j, gb, sem):
        pltpu.async_copy(node_flat.at[src_v.at[j]], gb, sem)

    def ewait(eb, sem):
        pltpu.make_async_copy(edge_flat.at[pl.ds(0, CE_), :], eb, sem).wait()

    def gwait(j, gb, sem):
        pltpu.make_async_copy(node_flat.at[src_v.at[j]], gb, sem).wait()

    def compute(eb, gb):
        @plsc.parallel_loop(0, CE_, step=1, unroll=2)
        def _row(r):
            for k in range(DH // 16):
                sl = pl.ds(k * 16, 16)
                gb[r, sl] = jnp.maximum(gb[r, sl] + eb[r, sl], 0.0)

    def block(o, carry0):
        pltpu.sync_copy(src_idx.at[c, s, o], src_v)
        pltpu.sync_copy(dst_idx.at[s, o], dst_v)

        estart(o, 0, ebuf, se0)
        gstart(0, gbuf, sg0)

        def pair(j2, carry):
            j0 = 2 * j2
            j1 = j0 + 1
            estart(o, j1, ebuf1, se1)
            gstart(j1, gbuf1, sg1)
            ewait(ebuf, se0)
            gwait(j0, gbuf, sg0)
            compute(ebuf, gbuf)
            pltpu.sync_copy(gbuf, agg_s.at[dst_v.at[j0]], add=True)

            @pl.when(j2 + 1 < BCH_MSG // 2)
            def _():
                estart(o, j0 + 2, ebuf, se0)
                gstart(j0 + 2, gbuf, sg0)

            ewait(ebuf1, se1)
            gwait(j1, gbuf1, sg1)
            compute(ebuf1, gbuf1)
            pltpu.sync_copy(gbuf1, agg_s.at[dst_v.at[j1]], add=True)
            return carry

        lax.fori_loop(0, BCH_MSG // 2, pair, 0, unroll=False)
        return carry0

    lax.fori_loop(0, NCH_MSG // BCH_MSG, block, 0, unroll=False)

    plsc.subcore_barrier()

    @pl.when(s < 15)
    def _():
        pltpu.sync_copy(agg_s.at[pl.ds(s * SZ_N, SZ_N), :],
                        agg_out.at[c, pl.ds(s * SZ_N, SZ_N), :])

    @pl.when(s == 15)
    def _():
        pltpu.sync_copy(agg_s.at[pl.ds(15 * SZ_N, SZ_N_LAST), :],
                        agg_out.at[c, pl.ds(15 * SZ_N, SZ_N_LAST), :])


def _sc_msg(node_flat, edge_flat, src_idx2, dst_idx, z128):
    f = pl.kernel(
        _msg_body,
        out_type=jax.ShapeDtypeStruct((NCORE, N, DH), jnp.float32),
        mesh=_mesh,
        scratch_types=[
            pltpu.VMEM((BCH_MSG, CE_), jnp.int32),
            pltpu.VMEM((BCH_MSG, CE_), jnp.int32),
            pltpu.VMEM((CE_, DH), jnp.float32),
            pltpu.VMEM((CE_, DH), jnp.float32),
            pltpu.VMEM((CE_, DH), jnp.float32),
            pltpu.VMEM((CE_, DH), jnp.float32),
            pltpu.VMEM_SHARED((N, DH), jnp.float32),
            pltpu.SemaphoreType.DMA,
            pltpu.SemaphoreType.DMA,
            pltpu.SemaphoreType.DMA,
            pltpu.SemaphoreType.DMA,
        ],
    )
    return f(node_flat, edge_flat, src_idx2, dst_idx, z128)


def _pool_body(node_flat, pc, ba_idx, b_idx, src_idx, dst_idx, hyp_idx,
               z128,
               a_sum, g_sum, r_part,
               ba_v, b_v, nbuf, srcv, dstv, hypv, pbuf, qbuf, pbuf1, qbuf1,
               actor_s, g_s, rel_s, sp0, sq0, sp1, sq1):
    c = lax.axis_index("c")
    s = lax.axis_index("s")
    w = c * NTEC + s

    @pl.when(s < 15)
    def _():
        pltpu.sync_copy(z128.at[pl.ds(0, SZ_A), :],
                        actor_s.at[pl.ds(s * SZ_A, SZ_A), :])
        pltpu.sync_copy(z128.at[pl.ds(0, SZ_R), :],
                        rel_s.at[pl.ds(s * SZ_R, SZ_R), :])

    @pl.when(s == 15)
    def _():
        pltpu.sync_copy(z128.at[pl.ds(0, SZ_A_LAST), :],
                        actor_s.at[pl.ds(15 * SZ_A, SZ_A_LAST), :])
        pltpu.sync_copy(z128.at[pl.ds(0, SZ_R_LAST), :],
                        rel_s.at[pl.ds(15 * SZ_R, SZ_R_LAST), :])

    @pl.when(s == 0)
    def _():
        pltpu.sync_copy(z128.at[pl.ds(0, NG), :], g_s)

    pltpu.sync_copy(ba_idx.at[pl.ds(s * 8, 8), :], ba_v)
    pltpu.sync_copy(b_idx.at[pl.ds(s * 8, 8), :], b_v)
    pltpu.sync_copy(src_idx.at[w], srcv)
    pltpu.sync_copy(dst_idx.at[w], dstv)
    pltpu.sync_copy(hyp_idx.at[w], hypv)

    plsc.subcore_barrier()

    def nchunk(j, carry):
        q = s * 8 + j

        @pl.when(q < NCH_POOL)
        def _():
            n0 = c * N + q * CN
            pltpu.sync_copy(node_flat.at[pl.ds(n0, CN), :], nbuf)
            pltpu.sync_copy(nbuf, actor_s.at[ba_v.at[j]], add=True)
            pltpu.sync_copy(nbuf, g_s.at[b_v.at[j]], add=True)

        return carry

    lax.fori_loop(0, 8, nchunk, 0, unroll=False)

    def rstart(j, pb_, qb_, semp, semq):
        pltpu.async_copy(pc.at[srcv.at[j]], pb_, semp)
        pltpu.async_copy(pc.at[dstv.at[j]], qb_, semq)

    def rwait(j, pb_, qb_, semp, semq):
        pltpu.make_async_copy(pc.at[srcv.at[j]], pb_, semp).wait()
        pltpu.make_async_copy(pc.at[dstv.at[j]], qb_, semq).wait()

    def radd(pb_, qb_):
        @plsc.parallel_loop(0, CR, step=1, unroll=2)
        def _row(r):
            for k in range(PJ // 16):
                pb_[r, pl.ds(k * 16, 16)] = (pb_[r, pl.ds(k * 16, 16)]
                                             + qb_[r, pl.ds(PJ + k * 16, 16)])

    rstart(0, pbuf, qbuf, sp0, sq0)

    def rpair(j2, carry):
        j0 = 2 * j2
        j1 = j0 + 1
        rstart(j1, pbuf1, qbuf1, sp1, sq1)
        rwait(j0, pbuf, qbuf, sp0, sq0)
        radd(pbuf, qbuf)
        pltpu.sync_copy(pbuf, rel_s.at[hypv.at[j0]], add=True)

        @pl.when(j2 + 1 < NCH_REL // 2)
        def _():
            rstart(j0 + 2, pbuf, qbuf, sp0, sq0)

        rwait(j1, pbuf1, qbuf1, sp1, sq1)
        radd(pbuf1, qbuf1)
        pltpu.sync_copy(pbuf1, rel_s.at[hypv.at[j1]], add=True)
        return carry

    lax.fori_loop(0, NCH_REL // 2, rpair, 0, unroll=False)

    plsc.subcore_barrier()

    @pl.when(s < 15)
    def _():
        pltpu.sync_copy(actor_s.at[pl.ds(s * SZ_A, SZ_A), :],
                        a_sum.at[c, pl.ds(s * SZ_A, SZ_A), :])
        pltpu.sync_copy(rel_s.at[pl.ds(s * SZ_R, SZ_R), :],
                        r_part.at[c, pl.ds(s * SZ_R, SZ_R), :])

    @pl.when(s == 15)
    def _():
        pltpu.sync_copy(actor_s.at[pl.ds(15 * SZ_A, SZ_A_LAST), :],
                        a_sum.at[c, pl.ds(15 * SZ_A, SZ_A_LAST), :])
        pltpu.sync_copy(rel_s.at[pl.ds(15 * SZ_R, SZ_R_LAST), :],
                        r_part.at[c, pl.ds(15 * SZ_R, SZ_R_LAST), :])

    @pl.when(s == 0)
    def _():
        pltpu.sync_copy(g_s, g_sum.at[c])


def _sc_pool(node_flat, pc, ba_idx, b_idx, src_idx, dst_idx, hyp_idx, z128):
    f = pl.kernel(
        _pool_body,
        out_type=[
            jax.ShapeDtypeStruct((NCORE, NA, DH), jnp.float32),
            jax.ShapeDtypeStruct((NCORE, NG, DH), jnp.float32),
            jax.ShapeDtypeStruct((NCORE, NH, DH), jnp.float32),
        ],
        mesh=_mesh,
        scratch_types=[
            pltpu.VMEM((8, CN), jnp.int32),
            pltpu.VMEM((8, CN), jnp.int32),
            pltpu.VMEM((CN, DH), jnp.float32),
            pltpu.VMEM((NCH_REL, CR), jnp.int32),
            pltpu.VMEM((NCH_REL, CR), jnp.int32),
            pltpu.VMEM((NCH_REL, CR), jnp.int32),
            pltpu.VMEM((CR, DH), jnp.float32),
            pltpu.VMEM((CR, DH), jnp.float32),
            pltpu.VMEM((CR, DH), jnp.float32),
            pltpu.VMEM((CR, DH), jnp.float32),
            pltpu.VMEM_SHARED((NA, DH), jnp.float32),
            pltpu.VMEM_SHARED((NG, DH), jnp.float32),
            pltpu.VMEM_SHARED((NH, DH), jnp.float32),
            pltpu.SemaphoreType.DMA,
            pltpu.SemaphoreType.DMA,
            pltpu.SemaphoreType.DMA,
            pltpu.SemaphoreType.DMA,
        ],
    )
    return f(node_flat, pc, ba_idx, b_idx, src_idx, dst_idx, hyp_idx, z128)


def _heads_kernel(a_sum, a_cnt, g_sum, g_cnt, r_part,
                  act_cids, sact_cids, ps_t, pa_t, actor_cids, h_label,
                  W_act, b_act, W_sact, b_sact, W_ps, b_ps, W_pa, b_pa,
                  W_actor, b_actor, b_relat, out):
    embed = a_sum[...] / jnp.clip(a_cnt[...], 1.0)
    g = g_sum[...] / jnp.clip(g_cnt[...], 1.0)

    def ce(logits, labels_col):
        m = jnp.max(logits, axis=-1, keepdims=True)
        z = logits - m
        logp = z - jnp.log(jnp.sum(jnp.exp(z), axis=-1, keepdims=True))
        iot = lax.broadcasted_iota(jnp.int32, logits.shape, 1)
        onehot = (iot == labels_col).astype(jnp.float32)
        return -jnp.sum(logp * onehot) / logits.shape[0]

    def bce(logits, t):
        v = jnp.clip(logits, 0.0) - logits * t + jnp.log(1.0 + jnp.exp(-jnp.abs(logits)))
        return jnp.sum(v) / (v.shape[0] * v.shape[1])

    logits_act = g @ W_act[...] + b_act[...]
    logits_sact = g @ W_sact[...] + b_sact[...]
    logits_ps = g @ W_ps[...] + b_ps[...]
    logits_pa = embed @ W_pa[...] + b_pa[...]
    logits_actor = embed @ W_actor[...] + b_actor[...]

    rp = r_part[...]
    rs = rp[0] + rp[1]
    rc = rs[:, NUM_RELAT:NUM_RELAT + 1]
    logits_relat = jnp.where(rc > 0.0,
                             rs[:, 0:NUM_RELAT] / jnp.clip(rc, 1.0) + b_relat[...],
                             0.0)

    loss = (ce(logits_act, act_cids[...])
            + ce(logits_sact, sact_cids[...])
            + bce(logits_ps, ps_t[...])
            + bce(logits_pa, pa_t[...])
            + ce(logits_actor, actor_cids[...])
            + ce(logits_relat, h_label[...]))
    out[...] = jnp.reshape(loss, (1, 1))


def _heads_loss(a_sum, a_cnt, g_sum, g_cnt, r_part,
                act_cids, sact_cids, ps_t, pa_t, actor_cids, h_label, p):
    args = (a_sum, a_cnt, g_sum, g_cnt, r_part,
            act_cids.reshape(NG, 1), sact_cids.reshape(NG, 1), ps_t, pa_t,
            actor_cids.reshape(NA, 1), h_label.reshape(NH, 1),
            p['W_act'], p['b_act'].reshape(1, -1),
            p['W_sact'], p['b_sact'].reshape(1, -1),
            p['W_ps'], p['b_ps'].reshape(1, -1),
            p['W_pa'], p['b_pa'].reshape(1, -1),
            p['W_actor'], p['b_actor'].reshape(1, -1),
            p['b_relat'].reshape(1, -1))
    out = pl.pallas_call(
        _heads_kernel,
        out_shape=jax.ShapeDtypeStruct((1, 1), jnp.float32),
    )(*args)
    return out.reshape(())


def kernel(x, edge_attr, edge_index, batch, batch_actor, hyperedge_ids,
           act_cids, sact_cids, ps_aact_cids, pa_aact_cids, actor_cids,
           hyperedge_label, params):
    p = params
    src = edge_index[0]
    dst = edge_index[1]

    src16 = src.reshape(NTEC, NCH_MSG // BCH_MSG, BCH_MSG, CE_)
    src16_2 = jnp.stack([src16, src16 + N])
    dst16 = dst.reshape(NTEC, NCH_MSG // BCH_MSG, BCH_MSG, CE_)
    src32 = src.reshape(NCORE * NTEC, NCH_REL, CR)
    dst32 = dst.reshape(NCORE * NTEC, NCH_REL, CR)
    hyp32 = hyperedge_ids.reshape(NCORE * NTEC, NCH_REL, CR)
    ba_idx = jnp.pad(batch_actor.reshape(NCH_POOL, CN), ((0, 3), (0, 0)))
    b_idx = jnp.pad(batch.reshape(NCH_POOL, CN), ((0, 3), (0, 0)))
    z128 = jnp.zeros((SZ_N, DH), jnp.float32)

    wtb = (jnp.zeros((D, DH), jnp.float32)
           .at[:, :NUM_RELAT].set(p['W_relat'][:D])
           .at[:, PJ:PJ + NUM_RELAT].set(p['W_relat'][D:]))
    crow = (jnp.zeros((1, DH), jnp.float32)
            .at[0, NUM_RELAT].set(0.5)
            .at[0, PJ + NUM_RELAT].set(0.5))

    node_st = _node_mlp(x, p['Wn1'], p['bn1'], p['Wn2'], p['bn2'])
    edge_st = _edge_mlp(edge_attr, p['We1'], p['be1'], p['We2'], p['be2'])
    node_flat = node_st.reshape(NCORE * N, DH)
    edge_flat = edge_st.reshape(NCORE * E, DH)

    agg1 = _sc_msg(node_flat, edge_flat, src16_2, dst16, z128)
    node_st = _layer(node_st, agg1, p['Wl1'], p['bl1'])
    node_flat = node_st.reshape(NCORE * N, DH)

    agg2 = _sc_msg(node_flat, edge_flat, src16_2, dst16, z128)
    node_st, pc = _layer2(node_st, agg2, p['Wl2'], p['bl2'], wtb, crow)
    node_flat = node_st.reshape(NCORE * N, DH)

    ca, cg = _counts(batch_actor, batch)
    a_sum, g_sum, r_part = _sc_pool(
        node_flat, pc, ba_idx, b_idx, src32, dst32, hyp32, z128)

    a_full = jnp.concatenate([a_sum[0], a_sum[1]], axis=1)
    g_full = jnp.concatenate([g_sum[0], g_sum[1]], axis=1)

    return _heads_loss(a_full, ca.reshape(NA, 1), g_full, cg.reshape(NG, 1),
                       r_part, act_cids, sact_cids, ps_aact_cids,
                       pa_aact_cids, actor_cids, hyperedge_label, p)


# speedup vs baseline: 1.1615x; 1.0002x over previous
import jax
import jax.numpy as jnp
from jax import lax
from jax.experimental import pallas as pl
from jax.experimental.pallas import tpu as pltpu
from jax.experimental.pallas import tpu_sc as plsc

N = 10000
E = 160000
D_IN = 512
D = 256
DH = 128
NG = 32
NA = 2000
NH = 4000
NUM_ACT = 20
NUM_SACT = 91
NUM_AACT = 17
NUM_ACTOR = 26
NUM_RELAT = 19

NTEC = 16
NCORE = 2
CE_ = 40
NCH_MSG = 250
BCH_MSG = 50
CN = 80
NCH_POOL = 125
CR = 50
NCH_REL = 100
PJ = 32

_mesh = plsc.VectorSubcoreMesh(core_axis_name="c", subcore_axis_name="s",
                               num_cores=NCORE, num_subcores=NTEC)


def _node_mlp_body(x_ref, w1_ref, b1_ref, w2_ref, b2_ref, out_ref):
    h = jnp.maximum(jnp.dot(x_ref[...].astype(jnp.bfloat16),
                            w1_ref[...].astype(jnp.bfloat16),
                            preferred_element_type=jnp.float32) + b1_ref[...], 0.0)
    res = jnp.dot(h.astype(jnp.bfloat16), w2_ref[...].astype(jnp.bfloat16),
                  preferred_element_type=jnp.float32) + b2_ref[...]
    out_ref[0] = res[:, :DH]
    out_ref[1] = res[:, DH:]


def _node_mlp(x, w1, b1, w2, b2):
    bn = 400
    grid = N // bn
    return pl.pallas_call(
        _node_mlp_body,
        grid=(grid,),
        in_specs=[
            pl.BlockSpec((bn, D_IN), lambda i: (i, 0)),
            pl.BlockSpec((D_IN, D), lambda i: (0, 0)),
            pl.BlockSpec((1, D), lambda i: (0, 0)),
            pl.BlockSpec((D, D), lambda i: (0, 0)),
            pl.BlockSpec((1, D), lambda i: (0, 0)),
        ],
        out_specs=pl.BlockSpec((NCORE, bn, DH), lambda i: (0, i, 0)),
        out_shape=jax.ShapeDtypeStruct((NCORE, N, DH), jnp.float32),
    )(x, w1, b1.reshape(1, D), w2, b2.reshape(1, D))


def _edge_mlp_body(e_ref, w1_ref, b1_ref, w2_ref, b2_ref, out_ref):
    h = jnp.maximum(jnp.dot(e_ref[...], w1_ref[...],
                            preferred_element_type=jnp.float32) + b1_ref[...], 0.0)
    res = jnp.dot(h.astype(jnp.bfloat16), w2_ref[...].astype(jnp.bfloat16),
                  preferred_element_type=jnp.float32) + b2_ref[...]
    out_ref[0] = res[:, :DH]
    out_ref[1] = res[:, DH:]


def _edge_mlp(ea, w1, b1, w2, b2):
    be = 2000
    grid = E // be
    return pl.pallas_call(
        _edge_mlp_body,
        grid=(grid,),
        in_specs=[
            pl.BlockSpec((be, 9), lambda i: (i, 0)),
            pl.BlockSpec((9, D), lambda i: (0, 0)),
            pl.BlockSpec((1, D), lambda i: (0, 0)),
            pl.BlockSpec((D, D), lambda i: (0, 0)),
            pl.BlockSpec((1, D), lambda i: (0, 0)),
        ],
        out_specs=pl.BlockSpec((NCORE, be, DH), lambda i: (0, i, 0)),
        out_shape=jax.ShapeDtypeStruct((NCORE, E, DH), jnp.float32),
    )(ea, w1, b1.reshape(1, D), w2, b2.reshape(1, D))


def _layer_body(n_ref, a_ref, w_ref, b_ref, out_ref):
    a0 = n_ref[0] + a_ref[0]
    a1 = n_ref[1] + a_ref[1]
    res = (jnp.dot(a0, w_ref[0:DH, :], preferred_element_type=jnp.float32)
           + jnp.dot(a1, w_ref[DH:, :], preferred_element_type=jnp.float32)
           + b_ref[...])
    res = jnp.maximum(res, 0.0)
    out_ref[0] = res[:, :DH]
    out_ref[1] = res[:, DH:]


def _layer(node_st, agg_st, w, b):
    bn = 400
    grid = N // bn
    return pl.pallas_call(
        _layer_body,
        grid=(grid,),
        in_specs=[
            pl.BlockSpec((NCORE, bn, DH), lambda i: (0, i, 0)),
            pl.BlockSpec((NCORE, bn, DH), lambda i: (0, i, 0)),
            pl.BlockSpec((D, D), lambda i: (0, 0)),
            pl.BlockSpec((1, D), lambda i: (0, 0)),
        ],
        out_specs=pl.BlockSpec((NCORE, bn, DH), lambda i: (0, i, 0)),
        out_shape=jax.ShapeDtypeStruct((NCORE, N, DH), jnp.float32),
    )(node_st, agg_st, w, b.reshape(1, D))


def _layer2_body(n_ref, a_ref, w_ref, b_ref, wtb_ref, c_ref,
                 out_ref, pc_ref):
    a0 = n_ref[0] + a_ref[0]
    a1 = n_ref[1] + a_ref[1]
    res = (jnp.dot(a0, w_ref[0:DH, :], preferred_element_type=jnp.float32)
           + jnp.dot(a1, w_ref[DH:, :], preferred_element_type=jnp.float32)
           + b_ref[...])
    res = jnp.maximum(res, 0.0)
    out_ref[0] = res[:, :DH]
    out_ref[1] = res[:, DH:]
    pc_ref[...] = jnp.dot(res, wtb_ref[...], preferred_element_type=jnp.float32) + c_ref[...]


def _layer2(node_st, agg_st, w, b, wtb, crow):
    bn = 400
    grid = N // bn
    return pl.pallas_call(
        _layer2_body,
        grid=(grid,),
        in_specs=[
            pl.BlockSpec((NCORE, bn, DH), lambda i: (0, i, 0)),
            pl.BlockSpec((NCORE, bn, DH), lambda i: (0, i, 0)),
            pl.BlockSpec((D, D), lambda i: (0, 0)),
            pl.BlockSpec((1, D), lambda i: (0, 0)),
            pl.BlockSpec((D, DH), lambda i: (0, 0)),
            pl.BlockSpec((1, DH), lambda i: (0, 0)),
        ],
        out_specs=[
            pl.BlockSpec((NCORE, bn, DH), lambda i: (0, i, 0)),
            pl.BlockSpec((bn, DH), lambda i: (i, 0)),
        ],
        out_shape=[
            jax.ShapeDtypeStruct((NCORE, N, DH), jnp.float32),
            jax.ShapeDtypeStruct((N, DH), jnp.float32),
        ],
    )(node_st, agg_st, w, b.reshape(1, D), wtb, crow)


def _counts_body(ba_ref, b_ref, ca_ref, cg_ref):
    i = pl.program_id(0)
    oh_a = (ba_ref[...] == lax.broadcasted_iota(jnp.int32, (200, NA), 1)
            ).astype(jnp.float32)
    oh_g = (b_ref[...] == lax.broadcasted_iota(jnp.int32, (200, NG), 1)
            ).astype(jnp.float32)
    pa = jnp.sum(oh_a, axis=0, keepdims=True)
    pg = jnp.sum(oh_g, axis=0, keepdims=True)

    @pl.when(i == 0)
    def _():
        ca_ref[...] = pa
        cg_ref[...] = pg

    @pl.when(i > 0)
    def _():
        ca_ref[...] = ca_ref[...] + pa
        cg_ref[...] = cg_ref[...] + pg


def _counts(batch_actor, batch):
    grid = N // 200
    return pl.pallas_call(
        _counts_body,
        grid=(grid,),
        in_specs=[
            pl.BlockSpec((200, 1), lambda i: (i, 0)),
            pl.BlockSpec((200, 1), lambda i: (i, 0)),
        ],
        out_specs=[
            pl.BlockSpec((1, NA), lambda i: (0, 0)),
            pl.BlockSpec((1, NG), lambda i: (0, 0)),
        ],
        out_shape=[
            jax.ShapeDtypeStruct((1, NA), jnp.float32),
            jax.ShapeDtypeStruct((1, NG), jnp.float32),
        ],
    )(batch_actor.reshape(N, 1), batch.reshape(N, 1))


def _striped(total):
    per = -(-total // NTEC)
    per = -(-per // 8) * 8
    last = total - 15 * per
    assert last > 0 and last % 8 == 0
    return per, last


SZ_N, SZ_N_LAST = _striped(N)
SZ_A, SZ_A_LAST = _striped(NA)
SZ_R, SZ_R_LAST = _striped(NH)


def _msg_body(node_flat, edge_flat, src_idx, dst_idx, z128, agg_out,
              src_v, dst_v, ebuf, gbuf, ebuf1, gbuf1, agg_s,
              se0, sg0, se1, sg1):
    c = lax.axis_index("c")
    s = lax.axis_index("s")

    @pl.when(s < 15)
    def _():
        pltpu.sync_copy(z128.at[pl.ds(0, SZ_N), :],
                        agg_s.at[pl.ds(s * SZ_N, SZ_N), :])

    @pl.when(s == 15)
    def _():
        pltpu.sync_copy(z128.at[pl.ds(0, SZ_N_LAST), :],
                        agg_s.at[pl.ds(15 * SZ_N, SZ_N_LAST), :])

    plsc.subcore_barrier()

    def estart(o, j, eb, sem):
        e0 = s * (NCH_MSG * CE_) + (o * BCH_MSG + j) * CE_
        pltpu.async_copy(edge_flat.at[pl.ds(c * E + e0, CE_), :], eb, sem)

    def gstart(j, gb, sem):
        pltpu.async_copy(node_flat.at[src_v.at[j]], gb, sem)

    def ewait(eb, sem):
        pltpu.make_async_copy(edge_flat.at[pl.ds(0, CE_), :], eb, sem).wait()

    def gwait(j, gb, sem):
        pltpu.make_async_copy(node_flat.at[src_v.at[j]], gb, sem).wait()

    def compute(eb, gb):
        @plsc.parallel_loop(0, CE_, step=1, unroll=2)
        def _row(r):
            for k in range(DH // 16):
                sl = pl.ds(k * 16, 16)
                gb[r, sl] = jnp.maximum(gb[r, sl] + eb[r, sl], 0.0)

    def block(o, carry0):
        pltpu.sync_copy(src_idx.at[c, s, o], src_v)
        pltpu.sync_copy(dst_idx.at[s, o], dst_v)

        estart(o, 0, ebuf, se0)
        gstart(0, gbuf, sg0)

        def pair(j2, carry):
            j0 = 2 * j2
            j1 = j0 + 1
            estart(o, j1, ebuf1, se1)
            gstart(j1, gbuf1, sg1)
            ewait(ebuf, se0)
            gwait(j0, gbuf, sg0)
            compute(ebuf, gbuf)
            pltpu.sync_copy(gbuf, agg_s.at[dst_v.at[j0]], add=True)

            @pl.when(j2 + 1 < BCH_MSG // 2)
            def _():
                estart(o, j0 + 2, ebuf, se0)
                gstart(j0 + 2, gbuf, sg0)

            ewait(ebuf1, se1)
            gwait(j1, gbuf1, sg1)
            compute(ebuf1, gbuf1)
            pltpu.sync_copy(gbuf1, agg_s.at[dst_v.at[j1]], add=True)
            return carry

        lax.fori_loop(0, BCH_MSG // 2, pair, 0, unroll=False)
        return carry0

    lax.fori_loop(0, NCH_MSG // BCH_MSG, block, 0, unroll=False)

    plsc.subcore_barrier()

    @pl.when(s < 15)
    def _():
        pltpu.sync_copy(agg_s.at[pl.ds(s * SZ_N, SZ_N), :],
                        agg_out.at[c, pl.ds(s * SZ_N, SZ_N), :])

    @pl.when(s == 15)
    def _():
        pltpu.sync_copy(agg_s.at[pl.ds(15 * SZ_N, SZ_N_LAST), :],
                        agg_out.at[c, pl.ds(15 * SZ_N, SZ_N_LAST), :])


def _sc_msg(node_flat, edge_flat, src_idx2, dst_idx, z128):
    f = pl.kernel(
        _msg_body,
        out_type=jax.ShapeDtypeStruct((NCORE, N, DH), jnp.float32),
        mesh=_mesh,
        scratch_types=[
            pltpu.VMEM((BCH_MSG, CE_), jnp.int32),
            pltpu.VMEM((BCH_MSG, CE_), jnp.int32),
            pltpu.VMEM((CE_, DH), jnp.float32),
            pltpu.VMEM((CE_, DH), jnp.float32),
            pltpu.VMEM((CE_, DH), jnp.float32),
            pltpu.VMEM((CE_, DH), jnp.float32),
            pltpu.VMEM_SHARED((N, DH), jnp.float32),
            pltpu.SemaphoreType.DMA,
            pltpu.SemaphoreType.DMA,
            pltpu.SemaphoreType.DMA,
            pltpu.SemaphoreType.DMA,
        ],
    )
    return f(node_flat, edge_flat, src_idx2, dst_idx, z128)


def _pool_body(node_flat, pc, ba_idx, b_idx, src_idx, dst_idx, hyp_idx,
               z128,
               a_sum, g_sum, r_part,
               ba_v, b_v, nbuf, srcv, dstv, hypv, pbuf, qbuf, pbuf1, qbuf1,
               actor_s, g_s, rel_s, sp0, sq0, sp1, sq1):
    c = lax.axis_index("c")
    s = lax.axis_index("s")
    w = c * NTEC + s

    @pl.when(s < 15)
    def _():
        pltpu.sync_copy(z128.at[pl.ds(0, SZ_A), :],
                        actor_s.at[pl.ds(s * SZ_A, SZ_A), :])
        pltpu.sync_copy(z128.at[pl.ds(0, SZ_R), :],
                        rel_s.at[pl.ds(s * SZ_R, SZ_R), :])

    @pl.when(s == 15)
    def _():
        pltpu.sync_copy(z128.at[pl.ds(0, SZ_A_LAST), :],
                        actor_s.at[pl.ds(15 * SZ_A, SZ_A_LAST), :])
        pltpu.sync_copy(z128.at[pl.ds(0, SZ_R_LAST), :],
                        rel_s.at[pl.ds(15 * SZ_R, SZ_R_LAST), :])

    @pl.when(s == 0)
    def _():
        pltpu.sync_copy(z128.at[pl.ds(0, NG), :], g_s)

    pltpu.sync_copy(ba_idx.at[pl.ds(s * 8, 8), :], ba_v)
    pltpu.sync_copy(b_idx.at[pl.ds(s * 8, 8), :], b_v)
    pltpu.sync_copy(src_idx.at[w], srcv)
    pltpu.sync_copy(dst_idx.at[w], dstv)
    pltpu.sync_copy(hyp_idx.at[w], hypv)

    plsc.subcore_barrier()

    def nchunk(j, carry):
        q = s * 8 + j

        @pl.when(q < NCH_POOL)
        def _():
            n0 = c * N + q * CN
            pltpu.sync_copy(node_flat.at[pl.ds(n0, CN), :], nbuf)
            pltpu.sync_copy(nbuf, actor_s.at[ba_v.at[j]], add=True)
            pltpu.sync_copy(nbuf, g_s.at[b_v.at[j]], add=True)

        return carry

    lax.fori_loop(0, 8, nchunk, 0, unroll=False)

    def rstart(j, pb_, qb_, semp, semq):
        pltpu.async_copy(pc.at[srcv.at[j]], pb_, semp)
        pltpu.async_copy(pc.at[dstv.at[j]], qb_, semq)

    def rwait(j, pb_, qb_, semp, semq):
        pltpu.make_async_copy(pc.at[srcv.at[j]], pb_, semp).wait()
        pltpu.make_async_copy(pc.at[dstv.at[j]], qb_, semq).wait()

    def radd(pb_, qb_):
        @plsc.parallel_loop(0, CR, step=1, unroll=2)
        def _row(r):
            for k in range(PJ // 16):
                pb_[r, pl.ds(k * 16, 16)] = (pb_[r, pl.ds(k * 16, 16)]
                                             + qb_[r, pl.ds(PJ + k * 16, 16)])

    rstart(0, pbuf, qbuf, sp0, sq0)

    def rpair(j2, carry):
        j0 = 2 * j2
        j1 = j0 + 1
        rstart(j1, pbuf1, qbuf1, sp1, sq1)
        rwait(j0, pbuf, qbuf, sp0, sq0)
        radd(pbuf, qbuf)
        pltpu.sync_copy(pbuf, rel_s.at[hypv.at[j0]], add=True)

        @pl.when(j2 + 1 < NCH_REL // 2)
        def _():
            rstart(j0 + 2, pbuf, qbuf, sp0, sq0)

        rwait(j1, pbuf1, qbuf1, sp1, sq1)
        radd(pbuf1, qbuf1)
        pltpu.sync_copy(pbuf1, rel_s.at[hypv.at[j1]], add=True)
        return carry

    lax.fori_loop(0, NCH_REL // 2, rpair, 0, unroll=False)

    plsc.subcore_barrier()

    @pl.when(s < 15)
    def _():
        pltpu.sync_copy(actor_s.at[pl.ds(s * SZ_A, SZ_A), :],
                        a_sum.at[c, pl.ds(s * SZ_A, SZ_A), :])
        pltpu.sync_copy(rel_s.at[pl.ds(s * SZ_R, SZ_R), :],
                        r_part.at[c, pl.ds(s * SZ_R, SZ_R), :])

    @pl.when(s == 15)
    def _():
        pltpu.sync_copy(actor_s.at[pl.ds(15 * SZ_A, SZ_A_LAST), :],
                        a_sum.at[c, pl.ds(15 * SZ_A, SZ_A_LAST), :])
        pltpu.sync_copy(rel_s.at[pl.ds(15 * SZ_R, SZ_R_LAST), :],
                        r_part.at[c, pl.ds(15 * SZ_R, SZ_R_LAST), :])

    @pl.when(s == 0)
    def _():
        pltpu.sync_copy(g_s, g_sum.at[c])


def _sc_pool(node_flat, pc, ba_idx, b_idx, src_idx, dst_idx, hyp_idx, z128):
    f = pl.kernel(
        _pool_body,
        out_type=[
            jax.ShapeDtypeStruct((NCORE, NA, DH), jnp.float32),
            jax.ShapeDtypeStruct((NCORE, NG, DH), jnp.float32),
            jax.ShapeDtypeStruct((NCORE, NH, DH), jnp.float32),
        ],
        mesh=_mesh,
        scratch_types=[
            pltpu.VMEM((8, CN), jnp.int32),
            pltpu.VMEM((8, CN), jnp.int32),
            pltpu.VMEM((CN, DH), jnp.float32),
            pltpu.VMEM((NCH_REL, CR), jnp.int32),
            pltpu.VMEM((NCH_REL, CR), jnp.int32),
            pltpu.VMEM((NCH_REL, CR), jnp.int32),
            pltpu.VMEM((CR, DH), jnp.float32),
            pltpu.VMEM((CR, DH), jnp.float32),
            pltpu.VMEM((CR, DH), jnp.float32),
            pltpu.VMEM((CR, DH), jnp.float32),
            pltpu.VMEM_SHARED((NA, DH), jnp.float32),
            pltpu.VMEM_SHARED((NG, DH), jnp.float32),
            pltpu.VMEM_SHARED((NH, DH), jnp.float32),
            pltpu.SemaphoreType.DMA,
            pltpu.SemaphoreType.DMA,
            pltpu.SemaphoreType.DMA,
            pltpu.SemaphoreType.DMA,
        ],
    )
    return f(node_flat, pc, ba_idx, b_idx, src_idx, dst_idx, hyp_idx, z128)


def _heads_kernel(a_sum, a_cnt, g_sum, g_cnt, r_part,
                  act_cids, sact_cids, ps_t, pa_t, actor_cids, h_label,
                  W_act, b_act, W_sact, b_sact, W_ps, b_ps, W_pa, b_pa,
                  W_actor, b_actor, b_relat, out):
    embed = a_sum[...] / jnp.clip(a_cnt[...], 1.0)
    g = g_sum[...] / jnp.clip(g_cnt[...], 1.0)

    def ce(logits, labels_col):
        m = jnp.max(logits, axis=-1, keepdims=True)
        z = logits - m
        logp = z - jnp.log(jnp.sum(jnp.exp(z), axis=-1, keepdims=True))
        iot = lax.broadcasted_iota(jnp.int32, logits.shape, 1)
        onehot = (iot == labels_col).astype(jnp.float32)
        return -jnp.sum(logp * onehot) / logits.shape[0]

    def bce(logits, t):
        v = jnp.clip(logits, 0.0) - logits * t + jnp.log(1.0 + jnp.exp(-jnp.abs(logits)))
        return jnp.sum(v) / (v.shape[0] * v.shape[1])

    logits_act = g @ W_act[...] + b_act[...]
    logits_sact = g @ W_sact[...] + b_sact[...]
    logits_ps = g @ W_ps[...] + b_ps[...]
    logits_pa = embed @ W_pa[...] + b_pa[...]
    logits_actor = embed @ W_actor[...] + b_actor[...]

    rp = r_part[...]
    rs = rp[0] + rp[1]
    rc = rs[:, NUM_RELAT:NUM_RELAT + 1]
    logits_relat = jnp.where(rc > 0.0,
                             rs[:, 0:NUM_RELAT] / jnp.clip(rc, 1.0) + b_relat[...],
                             0.0)

    loss = (ce(logits_act, act_cids[...])
            + ce(logits_sact, sact_cids[...])
            + bce(logits_ps, ps_t[...])
            + bce(logits_pa, pa_t[...])
            + ce(logits_actor, actor_cids[...])
            + ce(logits_relat, h_label[...]))
    out[...] = jnp.reshape(loss, (1, 1))


def _heads_loss(a_sum, a_cnt, g_sum, g_cnt, r_part,
                act_cids, sact_cids, ps_t, pa_t, actor_cids, h_label, p):
    args = (a_sum, a_cnt, g_sum, g_cnt, r_part,
            act_cids.reshape(NG, 1), sact_cids.reshape(NG, 1), ps_t, pa_t,
            actor_cids.reshape(NA, 1), h_label.reshape(NH, 1),
            p['W_act'], p['b_act'].reshape(1, -1),
            p['W_sact'], p['b_sact'].reshape(1, -1),
            p['W_ps'], p['b_ps'].reshape(1, -1),
            p['W_pa'], p['b_pa'].reshape(1, -1),
            p['W_actor'], p['b_actor'].reshape(1, -1),
            p['b_relat'].reshape(1, -1))
    out = pl.pallas_call(
        _heads_kernel,
        out_shape=jax.ShapeDtypeStruct((1, 1), jnp.float32),
    )(*args)
    return out.reshape(())


def kernel(x, edge_attr, edge_index, batch, batch_actor, hyperedge_ids,
           act_cids, sact_cids, ps_aact_cids, pa_aact_cids, actor_cids,
           hyperedge_label, params):
    p = params
    src = edge_index[0]
    dst = edge_index[1]

    src16 = src.reshape(NTEC, NCH_MSG // BCH_MSG, BCH_MSG, CE_)
    src16_2 = jnp.stack([src16, src16 + N])
    dst16 = dst.reshape(NTEC, NCH_MSG // BCH_MSG, BCH_MSG, CE_)
    src32 = src.reshape(NCORE * NTEC, NCH_REL, CR)
    dst32 = dst.reshape(NCORE * NTEC, NCH_REL, CR)
    hyp32 = hyperedge_ids.reshape(NCORE * NTEC, NCH_REL, CR)
    ba_idx = jnp.pad(batch_actor.reshape(NCH_POOL, CN), ((0, 3), (0, 0)))
    b_idx = jnp.pad(batch.reshape(NCH_POOL, CN), ((0, 3), (0, 0)))
    z128 = jnp.zeros((SZ_N, DH), jnp.float32)

    wtb = (jnp.zeros((D, DH), jnp.float32)
           .at[:, :NUM_RELAT].set(p['W_relat'][:D])
           .at[:, PJ:PJ + NUM_RELAT].set(p['W_relat'][D:]))
    crow = (jnp.zeros((1, DH), jnp.float32)
            .at[0, NUM_RELAT].set(0.5)
            .at[0, PJ + NUM_RELAT].set(0.5))

    node_st = _node_mlp(x, p['Wn1'], p['bn1'], p['Wn2'], p['bn2'])
    edge_st = _edge_mlp(edge_attr, p['We1'], p['be1'], p['We2'], p['be2'])
    node_flat = node_st.reshape(NCORE * N, DH)
    edge_flat = edge_st.reshape(NCORE * E, DH)

    agg1 = _sc_msg(node_flat, edge_flat, src16_2, dst16, z128)
    node_st = _layer(node_st, agg1, p['Wl1'], p['bl1'])
    node_flat = node_st.reshape(NCORE * N, DH)

    agg2 = _sc_msg(node_flat, edge_flat, src16_2, dst16, z128)
    node_st, pc = _layer2(node_st, agg2, p['Wl2'], p['bl2'], wtb, crow)
    node_flat = node_st.reshape(NCORE * N, DH)

    ca, cg = _counts(batch_actor, batch)
    a_sum, g_sum, r_part = _sc_pool(
        node_flat, pc, ba_idx, b_idx, src32, dst32, hyp32, z128)

    a_full = jnp.concatenate([a_sum[0], a_sum[1]], axis=1)
    g_full = jnp.concatenate([g_sum[0], g_sum[1]], axis=1)

    return _heads_loss(a_full, ca.reshape(NA, 1), g_full, cg.reshape(NG, 1),
                       r_part, act_cids, sact_cids, ps_aact_cids,
                       pa_aact_cids, actor_cids, hyperedge_label, p)


# speedup vs baseline: 1.1680x; 1.0056x over previous
import jax
import jax.numpy as jnp
from jax import lax
from jax.experimental import pallas as pl
from jax.experimental.pallas import tpu as pltpu
from jax.experimental.pallas import tpu_sc as plsc

N = 10000
E = 160000
D_IN = 512
D = 256
DH = 128
NG = 32
NA = 2000
NH = 4000
NUM_ACT = 20
NUM_SACT = 91
NUM_AACT = 17
NUM_ACTOR = 26
NUM_RELAT = 19

NTEC = 16
NCORE = 2
CE_ = 40
NCH_MSG = 250
BCH_MSG = 50
CN = 80
NCH_POOL = 125
CR = 50
NCH_REL = 100
PJ = 32

_mesh = plsc.VectorSubcoreMesh(core_axis_name="c", subcore_axis_name="s",
                               num_cores=NCORE, num_subcores=NTEC)


def _node_mlp_body(x_ref, w1_ref, b1_ref, w2_ref, b2_ref, out_ref):
    h = jnp.maximum(jnp.dot(x_ref[...].astype(jnp.bfloat16),
                            w1_ref[...].astype(jnp.bfloat16),
                            preferred_element_type=jnp.float32) + b1_ref[...], 0.0)
    res = jnp.dot(h.astype(jnp.bfloat16), w2_ref[...].astype(jnp.bfloat16),
                  preferred_element_type=jnp.float32) + b2_ref[...]
    out_ref[0] = res[:, :DH]
    out_ref[1] = res[:, DH:]


def _node_mlp(x, w1, b1, w2, b2):
    bn = 400
    grid = N // bn
    return pl.pallas_call(
        _node_mlp_body,
        grid=(grid,),
        in_specs=[
            pl.BlockSpec((bn, D_IN), lambda i: (i, 0)),
            pl.BlockSpec((D_IN, D), lambda i: (0, 0)),
            pl.BlockSpec((1, D), lambda i: (0, 0)),
            pl.BlockSpec((D, D), lambda i: (0, 0)),
            pl.BlockSpec((1, D), lambda i: (0, 0)),
        ],
        out_specs=pl.BlockSpec((NCORE, bn, DH), lambda i: (0, i, 0)),
        out_shape=jax.ShapeDtypeStruct((NCORE, N, DH), jnp.float32),
    )(x, w1, b1.reshape(1, D), w2, b2.reshape(1, D))


def _edge_mlp_body(e_ref, w1_ref, b1_ref, w2_ref, b2_ref, out_ref):
    h = jnp.maximum(jnp.dot(e_ref[...], w1_ref[...],
                            preferred_element_type=jnp.float32) + b1_ref[...], 0.0)
    res = jnp.dot(h.astype(jnp.bfloat16), w2_ref[...].astype(jnp.bfloat16),
                  preferred_element_type=jnp.float32) + b2_ref[...]
    out_ref[0] = res[:, :DH]
    out_ref[1] = res[:, DH:]


def _edge_mlp(ea, w1, b1, w2, b2):
    be = 2000
    grid = E // be
    return pl.pallas_call(
        _edge_mlp_body,
        grid=(grid,),
        in_specs=[
            pl.BlockSpec((be, 9), lambda i: (i, 0)),
            pl.BlockSpec((9, D), lambda i: (0, 0)),
            pl.BlockSpec((1, D), lambda i: (0, 0)),
            pl.BlockSpec((D, D), lambda i: (0, 0)),
            pl.BlockSpec((1, D), lambda i: (0, 0)),
        ],
        out_specs=pl.BlockSpec((NCORE, be, DH), lambda i: (0, i, 0)),
        out_shape=jax.ShapeDtypeStruct((NCORE, E, DH), jnp.float32),
    )(ea, w1, b1.reshape(1, D), w2, b2.reshape(1, D))


def _layer_body(n_ref, a_ref, w_ref, b_ref, out_ref):
    a0 = n_ref[0] + a_ref[0]
    a1 = n_ref[1] + a_ref[1]
    res = (jnp.dot(a0, w_ref[0:DH, :], preferred_element_type=jnp.float32)
           + jnp.dot(a1, w_ref[DH:, :], preferred_element_type=jnp.float32)
           + b_ref[...])
    res = jnp.maximum(res, 0.0)
    out_ref[0] = res[:, :DH]
    out_ref[1] = res[:, DH:]


def _layer(node_st, agg_st, w, b):
    bn = 400
    grid = N // bn
    return pl.pallas_call(
        _layer_body,
        grid=(grid,),
        in_specs=[
            pl.BlockSpec((NCORE, bn, DH), lambda i: (0, i, 0)),
            pl.BlockSpec((NCORE, bn, DH), lambda i: (0, i, 0)),
            pl.BlockSpec((D, D), lambda i: (0, 0)),
            pl.BlockSpec((1, D), lambda i: (0, 0)),
        ],
        out_specs=pl.BlockSpec((NCORE, bn, DH), lambda i: (0, i, 0)),
        out_shape=jax.ShapeDtypeStruct((NCORE, N, DH), jnp.float32),
    )(node_st, agg_st, w, b.reshape(1, D))


def _layer2_body(n_ref, a_ref, w_ref, b_ref, wtb_ref, c_ref,
                 out_ref, pc_ref):
    a0 = n_ref[0] + a_ref[0]
    a1 = n_ref[1] + a_ref[1]
    res = (jnp.dot(a0, w_ref[0:DH, :], preferred_element_type=jnp.float32)
           + jnp.dot(a1, w_ref[DH:, :], preferred_element_type=jnp.float32)
           + b_ref[...])
    res = jnp.maximum(res, 0.0)
    out_ref[0] = res[:, :DH]
    out_ref[1] = res[:, DH:]
    pc_ref[...] = jnp.dot(res, wtb_ref[...], preferred_element_type=jnp.float32) + c_ref[...]


def _layer2(node_st, agg_st, w, b, wtb, crow):
    bn = 400
    grid = N // bn
    return pl.pallas_call(
        _layer2_body,
        grid=(grid,),
        in_specs=[
            pl.BlockSpec((NCORE, bn, DH), lambda i: (0, i, 0)),
            pl.BlockSpec((NCORE, bn, DH), lambda i: (0, i, 0)),
            pl.BlockSpec((D, D), lambda i: (0, 0)),
            pl.BlockSpec((1, D), lambda i: (0, 0)),
            pl.BlockSpec((D, DH), lambda i: (0, 0)),
            pl.BlockSpec((1, DH), lambda i: (0, 0)),
        ],
        out_specs=[
            pl.BlockSpec((NCORE, bn, DH), lambda i: (0, i, 0)),
            pl.BlockSpec((bn, DH), lambda i: (i, 0)),
        ],
        out_shape=[
            jax.ShapeDtypeStruct((NCORE, N, DH), jnp.float32),
            jax.ShapeDtypeStruct((N, DH), jnp.float32),
        ],
    )(node_st, agg_st, w, b.reshape(1, D), wtb, crow)


def _counts_body(ba_ref, b_ref, ca_ref, cg_ref):
    i = pl.program_id(0)
    oh_a = (ba_ref[...] == lax.broadcasted_iota(jnp.int32, (200, NA), 1)
            ).astype(jnp.float32)
    oh_g = (b_ref[...] == lax.broadcasted_iota(jnp.int32, (200, NG), 1)
            ).astype(jnp.float32)
    pa = jnp.sum(oh_a, axis=0, keepdims=True)
    pg = jnp.sum(oh_g, axis=0, keepdims=True)

    @pl.when(i == 0)
    def _():
        ca_ref[...] = pa
        cg_ref[...] = pg

    @pl.when(i > 0)
    def _():
        ca_ref[...] = ca_ref[...] + pa
        cg_ref[...] = cg_ref[...] + pg


def _counts(batch_actor, batch):
    grid = N // 200
    return pl.pallas_call(
        _counts_body,
        grid=(grid,),
        in_specs=[
            pl.BlockSpec((200, 1), lambda i: (i, 0)),
            pl.BlockSpec((200, 1), lambda i: (i, 0)),
        ],
        out_specs=[
            pl.BlockSpec((1, NA), lambda i: (0, 0)),
            pl.BlockSpec((1, NG), lambda i: (0, 0)),
        ],
        out_shape=[
            jax.ShapeDtypeStruct((1, NA), jnp.float32),
            jax.ShapeDtypeStruct((1, NG), jnp.float32),
        ],
    )(batch_actor.reshape(N, 1), batch.reshape(N, 1))


def _striped(total):
    per = -(-total // NTEC)
    per = -(-per // 8) * 8
    last = total - 15 * per
    assert last > 0 and last % 8 == 0
    return per, last


SZ_N, SZ_N_LAST = _striped(N)
SZ_A, SZ_A_LAST = _striped(NA)
SZ_R, SZ_R_LAST = _striped(NH)


def _msg_body(node_flat, edge_flat, src_idx, dst_idx, z128, agg_out,
              src_v, dst_v, ebuf, gbuf, ebuf1, gbuf1, agg_s,
              se0, sg0, se1, sg1):
    c = lax.axis_index("c")
    s = lax.axis_index("s")

    @pl.when(s < 15)
    def _():
        pltpu.sync_copy(z128.at[pl.ds(0, SZ_N), :],
                        agg_s.at[pl.ds(s * SZ_N, SZ_N), :])

    @pl.when(s == 15)
    def _():
        pltpu.sync_copy(z128.at[pl.ds(0, SZ_N_LAST), :],
                        agg_s.at[pl.ds(15 * SZ_N, SZ_N_LAST), :])

    plsc.subcore_barrier()

    def estart(o, j, eb, sem):
        e0 = s * (NCH_MSG * CE_) + (o * BCH_MSG + j) * CE_
        pltpu.async_copy(edge_flat.at[pl.ds(c * E + e0, CE_), :], eb, sem)

    def gstart(j, gb, sem):
        pltpu.async_copy(node_flat.at[src_v.at[j]], gb, sem)

    def ewait(eb, sem):
        pltpu.make_async_copy(edge_flat.at[pl.ds(0, CE_), :], eb, sem).wait()

    def gwait(j, gb, sem):
        pltpu.make_async_copy(node_flat.at[src_v.at[j]], gb, sem).wait()

    def compute(eb, gb):
        @plsc.parallel_loop(0, CE_, step=1, unroll=4)
        def _row(r):
            for k in range(DH // 16):
                sl = pl.ds(k * 16, 16)
                gb[r, sl] = jnp.maximum(gb[r, sl] + eb[r, sl], 0.0)

    def block(o, carry0):
        pltpu.sync_copy(src_idx.at[c, s, o], src_v)
        pltpu.sync_copy(dst_idx.at[s, o], dst_v)

        estart(o, 0, ebuf, se0)
        gstart(0, gbuf, sg0)

        def pair(j2, carry):
            j0 = 2 * j2
            j1 = j0 + 1
            estart(o, j1, ebuf1, se1)
            gstart(j1, gbuf1, sg1)
            ewait(ebuf, se0)
            gwait(j0, gbuf, sg0)
            compute(ebuf, gbuf)
            pltpu.sync_copy(gbuf, agg_s.at[dst_v.at[j0]], add=True)

            @pl.when(j2 + 1 < BCH_MSG // 2)
            def _():
                estart(o, j0 + 2, ebuf, se0)
                gstart(j0 + 2, gbuf, sg0)

            ewait(ebuf1, se1)
            gwait(j1, gbuf1, sg1)
            compute(ebuf1, gbuf1)
            pltpu.sync_copy(gbuf1, agg_s.at[dst_v.at[j1]], add=True)
            return carry

        lax.fori_loop(0, BCH_MSG // 2, pair, 0, unroll=False)
        return carry0

    lax.fori_loop(0, NCH_MSG // BCH_MSG, block, 0, unroll=False)

    plsc.subcore_barrier()

    @pl.when(s < 15)
    def _():
        pltpu.sync_copy(agg_s.at[pl.ds(s * SZ_N, SZ_N), :],
                        agg_out.at[c, pl.ds(s * SZ_N, SZ_N), :])

    @pl.when(s == 15)
    def _():
        pltpu.sync_copy(agg_s.at[pl.ds(15 * SZ_N, SZ_N_LAST), :],
                        agg_out.at[c, pl.ds(15 * SZ_N, SZ_N_LAST), :])


def _sc_msg(node_flat, edge_flat, src_idx2, dst_idx, z128):
    f = pl.kernel(
        _msg_body,
        out_type=jax.ShapeDtypeStruct((NCORE, N, DH), jnp.float32),
        mesh=_mesh,
        scratch_types=[
            pltpu.VMEM((BCH_MSG, CE_), jnp.int32),
            pltpu.VMEM((BCH_MSG, CE_), jnp.int32),
            pltpu.VMEM((CE_, DH), jnp.float32),
            pltpu.VMEM((CE_, DH), jnp.float32),
            pltpu.VMEM((CE_, DH), jnp.float32),
            pltpu.VMEM((CE_, DH), jnp.float32),
            pltpu.VMEM_SHARED((N, DH), jnp.float32),
            pltpu.SemaphoreType.DMA,
            pltpu.SemaphoreType.DMA,
            pltpu.SemaphoreType.DMA,
            pltpu.SemaphoreType.DMA,
        ],
    )
    return f(node_flat, edge_flat, src_idx2, dst_idx, z128)


def _pool_body(node_flat, pc, ba_idx, b_idx, src_idx, dst_idx, hyp_idx,
               z128,
               a_sum, g_sum, r_part,
               ba_v, b_v, nbuf, srcv, dstv, hypv, pbuf, qbuf, pbuf1, qbuf1,
               actor_s, g_s, rel_s, sp0, sq0, sp1, sq1):
    c = lax.axis_index("c")
    s = lax.axis_index("s")
    w = c * NTEC + s

    @pl.when(s < 15)
    def _():
        pltpu.sync_copy(z128.at[pl.ds(0, SZ_A), :],
                        actor_s.at[pl.ds(s * SZ_A, SZ_A), :])
        pltpu.sync_copy(z128.at[pl.ds(0, SZ_R), :],
                        rel_s.at[pl.ds(s * SZ_R, SZ_R), :])

    @pl.when(s == 15)
    def _():
        pltpu.sync_copy(z128.at[pl.ds(0, SZ_A_LAST), :],
                        actor_s.at[pl.ds(15 * SZ_A, SZ_A_LAST), :])
        pltpu.sync_copy(z128.at[pl.ds(0, SZ_R_LAST), :],
                        rel_s.at[pl.ds(15 * SZ_R, SZ_R_LAST), :])

    @pl.when(s == 0)
    def _():
        pltpu.sync_copy(z128.at[pl.ds(0, NG), :], g_s)

    pltpu.sync_copy(ba_idx.at[pl.ds(s * 8, 8), :], ba_v)
    pltpu.sync_copy(b_idx.at[pl.ds(s * 8, 8), :], b_v)
    pltpu.sync_copy(src_idx.at[w], srcv)
    pltpu.sync_copy(dst_idx.at[w], dstv)
    pltpu.sync_copy(hyp_idx.at[w], hypv)

    plsc.subcore_barrier()

    def nchunk(j, carry):
        q = s * 8 + j

        @pl.when(q < NCH_POOL)
        def _():
            n0 = c * N + q * CN
            pltpu.sync_copy(node_flat.at[pl.ds(n0, CN), :], nbuf)
            pltpu.sync_copy(nbuf, actor_s.at[ba_v.at[j]], add=True)
            pltpu.sync_copy(nbuf, g_s.at[b_v.at[j]], add=True)

        return carry

    lax.fori_loop(0, 8, nchunk, 0, unroll=False)

    def rstart(j, pb_, qb_, semp, semq):
        pltpu.async_copy(pc.at[srcv.at[j]], pb_, semp)
        pltpu.async_copy(pc.at[dstv.at[j]], qb_, semq)

    def rwait(j, pb_, qb_, semp, semq):
        pltpu.make_async_copy(pc.at[srcv.at[j]], pb_, semp).wait()
        pltpu.make_async_copy(pc.at[dstv.at[j]], qb_, semq).wait()

    def radd(pb_, qb_):
        @plsc.parallel_loop(0, CR, step=1, unroll=2)
        def _row(r):
            for k in range(PJ // 16):
                pb_[r, pl.ds(k * 16, 16)] = (pb_[r, pl.ds(k * 16, 16)]
                                             + qb_[r, pl.ds(PJ + k * 16, 16)])

    rstart(0, pbuf, qbuf, sp0, sq0)

    def rpair(j2, carry):
        j0 = 2 * j2
        j1 = j0 + 1
        rstart(j1, pbuf1, qbuf1, sp1, sq1)
        rwait(j0, pbuf, qbuf, sp0, sq0)
        radd(pbuf, qbuf)
        pltpu.sync_copy(pbuf, rel_s.at[hypv.at[j0]], add=True)

        @pl.when(j2 + 1 < NCH_REL // 2)
        def _():
            rstart(j0 + 2, pbuf, qbuf, sp0, sq0)

        rwait(j1, pbuf1, qbuf1, sp1, sq1)
        radd(pbuf1, qbuf1)
        pltpu.sync_copy(pbuf1, rel_s.at[hypv.at[j1]], add=True)
        return carry

    lax.fori_loop(0, NCH_REL // 2, rpair, 0, unroll=False)

    plsc.subcore_barrier()

    @pl.when(s < 15)
    def _():
        pltpu.sync_copy(actor_s.at[pl.ds(s * SZ_A, SZ_A), :],
                        a_sum.at[c, pl.ds(s * SZ_A, SZ_A), :])
        pltpu.sync_copy(rel_s.at[pl.ds(s * SZ_R, SZ_R), :],
                        r_part.at[c, pl.ds(s * SZ_R, SZ_R), :])

    @pl.when(s == 15)
    def _():
        pltpu.sync_copy(actor_s.at[pl.ds(15 * SZ_A, SZ_A_LAST), :],
                        a_sum.at[c, pl.ds(15 * SZ_A, SZ_A_LAST), :])
        pltpu.sync_copy(rel_s.at[pl.ds(15 * SZ_R, SZ_R_LAST), :],
                        r_part.at[c, pl.ds(15 * SZ_R, SZ_R_LAST), :])

    @pl.when(s == 0)
    def _():
        pltpu.sync_copy(g_s, g_sum.at[c])


def _sc_pool(node_flat, pc, ba_idx, b_idx, src_idx, dst_idx, hyp_idx, z128):
    f = pl.kernel(
        _pool_body,
        out_type=[
            jax.ShapeDtypeStruct((NCORE, NA, DH), jnp.float32),
            jax.ShapeDtypeStruct((NCORE, NG, DH), jnp.float32),
            jax.ShapeDtypeStruct((NCORE, NH, DH), jnp.float32),
        ],
        mesh=_mesh,
        scratch_types=[
            pltpu.VMEM((8, CN), jnp.int32),
            pltpu.VMEM((8, CN), jnp.int32),
            pltpu.VMEM((CN, DH), jnp.float32),
            pltpu.VMEM((NCH_REL, CR), jnp.int32),
            pltpu.VMEM((NCH_REL, CR), jnp.int32),
            pltpu.VMEM((NCH_REL, CR), jnp.int32),
            pltpu.VMEM((CR, DH), jnp.float32),
            pltpu.VMEM((CR, DH), jnp.float32),
            pltpu.VMEM((CR, DH), jnp.float32),
            pltpu.VMEM((CR, DH), jnp.float32),
            pltpu.VMEM_SHARED((NA, DH), jnp.float32),
            pltpu.VMEM_SHARED((NG, DH), jnp.float32),
            pltpu.VMEM_SHARED((NH, DH), jnp.float32),
            pltpu.SemaphoreType.DMA,
            pltpu.SemaphoreType.DMA,
            pltpu.SemaphoreType.DMA,
            pltpu.SemaphoreType.DMA,
        ],
    )
    return f(node_flat, pc, ba_idx, b_idx, src_idx, dst_idx, hyp_idx, z128)


def _heads_kernel(a_sum, a_cnt, g_sum, g_cnt, r_part,
                  act_cids, sact_cids, ps_t, pa_t, actor_cids, h_label,
                  W_act, b_act, W_sact, b_sact, W_ps, b_ps, W_pa, b_pa,
                  W_actor, b_actor, b_relat, out):
    embed = a_sum[...] / jnp.clip(a_cnt[...], 1.0)
    g = g_sum[...] / jnp.clip(g_cnt[...], 1.0)

    def ce(logits, labels_col):
        m = jnp.max(logits, axis=-1, keepdims=True)
        z = logits - m
        logp = z - jnp.log(jnp.sum(jnp.exp(z), axis=-1, keepdims=True))
        iot = lax.broadcasted_iota(jnp.int32, logits.shape, 1)
        onehot = (iot == labels_col).astype(jnp.float32)
        return -jnp.sum(logp * onehot) / logits.shape[0]

    def bce(logits, t):
        v = jnp.clip(logits, 0.0) - logits * t + jnp.log(1.0 + jnp.exp(-jnp.abs(logits)))
        return jnp.sum(v) / (v.shape[0] * v.shape[1])

    logits_act = g @ W_act[...] + b_act[...]
    logits_sact = g @ W_sact[...] + b_sact[...]
    logits_ps = g @ W_ps[...] + b_ps[...]
    logits_pa = embed @ W_pa[...] + b_pa[...]
    logits_actor = embed @ W_actor[...] + b_actor[...]

    rp = r_part[...]
    rs = rp[0] + rp[1]
    rc = rs[:, NUM_RELAT:NUM_RELAT + 1]
    logits_relat = jnp.where(rc > 0.0,
                             rs[:, 0:NUM_RELAT] / jnp.clip(rc, 1.0) + b_relat[...],
                             0.0)

    loss = (ce(logits_act, act_cids[...])
            + ce(logits_sact, sact_cids[...])
            + bce(logits_ps, ps_t[...])
            + bce(logits_pa, pa_t[...])
            + ce(logits_actor, actor_cids[...])
            + ce(logits_relat, h_label[...]))
    out[...] = jnp.reshape(loss, (1, 1))


def _heads_loss(a_sum, a_cnt, g_sum, g_cnt, r_part,
                act_cids, sact_cids, ps_t, pa_t, actor_cids, h_label, p):
    args = (a_sum, a_cnt, g_sum, g_cnt, r_part,
            act_cids.reshape(NG, 1), sact_cids.reshape(NG, 1), ps_t, pa_t,
            actor_cids.reshape(NA, 1), h_label.reshape(NH, 1),
            p['W_act'], p['b_act'].reshape(1, -1),
            p['W_sact'], p['b_sact'].reshape(1, -1),
            p['W_ps'], p['b_ps'].reshape(1, -1),
            p['W_pa'], p['b_pa'].reshape(1, -1),
            p['W_actor'], p['b_actor'].reshape(1, -1),
            p['b_relat'].reshape(1, -1))
    out = pl.pallas_call(
        _heads_kernel,
        out_shape=jax.ShapeDtypeStruct((1, 1), jnp.float32),
    )(*args)
    return out.reshape(())


def kernel(x, edge_attr, edge_index, batch, batch_actor, hyperedge_ids,
           act_cids, sact_cids, ps_aact_cids, pa_aact_cids, actor_cids,
           hyperedge_label, params):
    p = params
    src = edge_index[0]
    dst = edge_index[1]

    src16 = src.reshape(NTEC, NCH_MSG // BCH_MSG, BCH_MSG, CE_)
    src16_2 = jnp.stack([src16, src16 + N])
    dst16 = dst.reshape(NTEC, NCH_MSG // BCH_MSG, BCH_MSG, CE_)
    src32 = src.reshape(NCORE * NTEC, NCH_REL, CR)
    dst32 = dst.reshape(NCORE * NTEC, NCH_REL, CR)
    hyp32 = hyperedge_ids.reshape(NCORE * NTEC, NCH_REL, CR)
    ba_idx = jnp.pad(batch_actor.reshape(NCH_POOL, CN), ((0, 3), (0, 0)))
    b_idx = jnp.pad(batch.reshape(NCH_POOL, CN), ((0, 3), (0, 0)))
    z128 = jnp.zeros((SZ_N, DH), jnp.float32)

    wtb = (jnp.zeros((D, DH), jnp.float32)
           .at[:, :NUM_RELAT].set(p['W_relat'][:D])
           .at[:, PJ:PJ + NUM_RELAT].set(p['W_relat'][D:]))
    crow = (jnp.zeros((1, DH), jnp.float32)
            .at[0, NUM_RELAT].set(0.5)
            .at[0, PJ + NUM_RELAT].set(0.5))

    node_st = _node_mlp(x, p['Wn1'], p['bn1'], p['Wn2'], p['bn2'])
    edge_st = _edge_mlp(edge_attr, p['We1'], p['be1'], p['We2'], p['be2'])
    node_flat = node_st.reshape(NCORE * N, DH)
    edge_flat = edge_st.reshape(NCORE * E, DH)

    agg1 = _sc_msg(node_flat, edge_flat, src16_2, dst16, z128)
    node_st = _layer(node_st, agg1, p['Wl1'], p['bl1'])
    node_flat = node_st.reshape(NCORE * N, DH)

    agg2 = _sc_msg(node_flat, edge_flat, src16_2, dst16, z128)
    node_st, pc = _layer2(node_st, agg2, p['Wl2'], p['bl2'], wtb, crow)
    node_flat = node_st.reshape(NCORE * N, DH)

    ca, cg = _counts(batch_actor, batch)
    a_sum, g_sum, r_part = _sc_pool(
        node_flat, pc, ba_idx, b_idx, src32, dst32, hyp32, z128)

    a_full = jnp.concatenate([a_sum[0], a_sum[1]], axis=1)
    g_full = jnp.concatenate([g_sum[0], g_sum[1]], axis=1)

    return _heads_loss(a_full, ca.reshape(NA, 1), g_full, cg.reshape(NG, 1),
                       r_part, act_cids, sact_cids, ps_aact_cids,
                       pa_aact_cids, actor_cids, hyperedge_label, p)
